# baseline (speedup 1.0000x reference)
import functools

import jax
import jax.numpy as jnp
from jax import lax
from jax.experimental import pallas as pl
from jax.experimental.pallas import tpu as pltpu
from jax.experimental.pallas import tpu_sc as plsc

B = 16384
EMB = 32
NTAB = 5
NC = 2
NS = 16
NW = NC * NS
B_PER_W = B // NW


def _sc_lookup_concat(authdir, year, actor, rated, genre,
                      W_authdir, W_year, W_actor, W_rated, W_genre):
    mesh = plsc.VectorSubcoreMesh(core_axis_name="c", subcore_axis_name="s",
                                  num_cores=NC, num_subcores=NS)

    @functools.partial(
        pl.kernel,
        mesh=mesh,
        out_type=jax.ShapeDtypeStruct((B, NTAB * EMB), jnp.float32),
        scratch_types=(
            [pltpu.VMEM((B_PER_W,), jnp.int32) for _ in range(NTAB)]
            + [pltpu.VMEM((B_PER_W, EMB), jnp.float32) for _ in range(NTAB)]
            + [pltpu.SemaphoreType.DMA, pltpu.SemaphoreType.DMA]
        ),
        compiler_params=pltpu.CompilerParams(use_tc_tiling_on_sc=False),
    )
    def body(a_i, y_i, ac_i, r_i, g_i, Wa, Wy, Wac, Wr, Wg, out,
             i0, i1, i2, i3, i4, r0, r1, r2, r3, r4, sem_i, sem_g):
        wid = lax.axis_index("s") * NC + lax.axis_index("c")
        base = wid * B_PER_W
        idx_hbm = [a_i, y_i, ac_i, r_i, g_i]
        tabs = [Wa, Wy, Wac, Wr, Wg]
        idx_v = [i0, i1, i2, i3, i4]
        row_v = [r0, r1, r2, r3, r4]

        idx_copies = [
            pltpu.async_copy(idx_hbm[c].at[pl.ds(base, B_PER_W)], idx_v[c], sem_i)
            for c in range(NTAB)
        ]
        gathers = []
        for c in range(NTAB):
            idx_copies[c].wait()
            gathers.append(pltpu.async_copy(tabs[c].at[idx_v[c]], row_v[c], sem_g))
        for c in range(NTAB):
            gathers[c].wait()
            pltpu.sync_copy(row_v[c],
                            out.at[pl.ds(base, B_PER_W), pl.ds(c * EMB, EMB)])

    return body(authdir, year, actor, rated, genre,
                W_authdir, W_year, W_actor, W_rated, W_genre)


def kernel(authdir, year, actor, rated, genre,
           W_authdir, W_year, W_actor, W_rated, W_genre):
    return _sc_lookup_concat(authdir, year, actor, rated, genre,
                             W_authdir, W_year, W_actor, W_rated, W_genre)
